# baseline (speedup 1.0000x reference)
import jax
import jax.numpy as jnp
from jax import lax
from jax.experimental import pallas as pl
from jax.experimental.pallas import tpu as pltpu
from jax.experimental.pallas import tpu_sc as plsc

N = 10000
E = 320000
D = 128
H = 64
C = 30
CP = 32
REG = 0.01

NC = 2
NS = 16
NW = NC * NS
NP = 10240
RPT = NP // NS
EPT = 10240
EPAD = NW * EPT
NCH = 32
CHK = 320

F32 = jnp.float32


def _mesh():
    return plsc.VectorSubcoreMesh(
        core_axis_name="c", subcore_axis_name="s", num_cores=NC, num_subcores=NS
    )


def _deg_body(dst_hbm, ea_hbm, z1_hbm, out_hbm, deg_sh, idx_v, val_v):
    cid = lax.axis_index("c")
    sid = lax.axis_index("s")
    wid = cid * NS + sid
    r0 = sid * RPT
    pltpu.sync_copy(z1_hbm.at[pl.ds(r0, RPT)], deg_sh.at[pl.ds(r0, RPT)])
    pltpu.sync_copy(dst_hbm.at[wid], idx_v)
    pltpu.sync_copy(ea_hbm.at[wid], val_v)
    plsc.subcore_barrier()

    def chunk(j, carry):
        pltpu.sync_copy(val_v.at[j], deg_sh.at[idx_v.at[j]], add=True)
        return carry

    lax.fori_loop(0, NCH, chunk, 0)
    plsc.subcore_barrier()
    pltpu.sync_copy(deg_sh.at[pl.ds(r0, RPT)], out_hbm.at[cid, pl.ds(r0, RPT)])


_deg_call = pl.kernel(
    _deg_body,
    out_type=jax.ShapeDtypeStruct((NC, NP), F32),
    mesh=_mesh(),
    compiler_params=pltpu.CompilerParams(use_tc_tiling_on_sc=False),
    scratch_types=[
        pltpu.VMEM_SHARED((NP,), F32),
        pltpu.VMEM((NCH, CHK), jnp.int32),
        pltpu.VMEM((NCH, CHK), F32),
    ],
)


def _make_acc(Dw):
    nq = Dw // 16

    def body(tab_hbm, src_hbm, dst_hbm, ea_hbm, zt_hbm, out_hbm,
             acc_sh, srcv, dstv, eav, rows0, rows1, g0, g1, w0, w1):
        cid = lax.axis_index("c")
        sid = lax.axis_index("s")
        wid = cid * NS + sid
        r0 = sid * RPT
        pltpu.sync_copy(zt_hbm.at[pl.ds(r0, RPT)], acc_sh.at[pl.ds(r0, RPT)])
        pltpu.sync_copy(src_hbm.at[wid], srcv)
        pltpu.sync_copy(dst_hbm.at[wid], dstv)
        pltpu.sync_copy(ea_hbm.at[wid], eav)
        plsc.subcore_barrier()

        rows = (rows0, rows1)
        gsem = (g0, g1)
        wsem = (w0, w1)

        def g_start(b, j):
            pltpu.async_copy(tab_hbm.at[srcv.at[j]], rows[b], gsem[b])

        def g_wait(b, j):
            pltpu.make_async_copy(tab_hbm.at[srcv.at[j]], rows[b], gsem[b]).wait()

        def w_start(b, j):
            pltpu.async_copy(rows[b], acc_sh.at[dstv.at[j]], wsem[b], add=True)

        def w_wait(b, j):
            pltpu.make_async_copy(rows[b], acc_sh.at[dstv.at[j]], wsem[b]).wait()

        def scale(j, rbuf):
            @plsc.parallel_loop(0, CHK // 16, unroll=4)
            def grp(t):
                wv = eav[j, pl.ds(t * 16, 16)]
                e0 = t * 16
                for l in range(16):
                    w = wv[l]
                    for q in range(nq):
                        sl = pl.ds(q * 16, 16)
                        rbuf[e0 + l, sl] = rbuf[e0 + l, sl] * w

        pltpu.sync_copy(tab_hbm.at[srcv.at[0]], rows[0])
        scale(0, rows[0])
        w_start(0, 0)
        g_start(1, 1)

        def slot(b, j):
            g_wait(b, j)
            scale(j, rows[b])
            w_wait(1 - b, j - 1)
            w_start(b, j)
            g_start(1 - b, j + 1)

        def pair(jj, carry):
            slot(1, 2 * jj + 1)
            slot(0, 2 * jj + 2)
            return carry

        lax.fori_loop(0, (NCH - 2) // 2, pair, 0)
        jL = NCH - 1
        g_wait(1, jL)
        scale(jL, rows[1])
        w_wait(0, jL - 1)
        w_start(1, jL)
        w_wait(1, jL)
        plsc.subcore_barrier()
        pltpu.sync_copy(acc_sh.at[pl.ds(r0, RPT)], out_hbm.at[cid, pl.ds(r0, RPT)])

    return pl.kernel(
        body,
        out_type=jax.ShapeDtypeStruct((NC, NP, Dw), F32),
        mesh=_mesh(),
        compiler_params=pltpu.CompilerParams(use_tc_tiling_on_sc=False),
        scratch_types=[
            pltpu.VMEM_SHARED((NP, Dw), F32),
            pltpu.VMEM((NCH, CHK), jnp.int32),
            pltpu.VMEM((NCH, CHK), jnp.int32),
            pltpu.VMEM((NCH, CHK), F32),
            pltpu.VMEM((CHK, Dw), F32),
            pltpu.VMEM((CHK, Dw), F32),
            pltpu.SemaphoreType.DMA,
            pltpu.SemaphoreType.DMA,
            pltpu.SemaphoreType.DMA,
            pltpu.SemaphoreType.DMA,
        ],
    )


_acc64_call = _make_acc(H)
_acc32_call = _make_acc(CP)


def _loss_body(fx_hbm, src_hbm, dst_hbm, ea_hbm, out_hbm,
               fx_sh, srcv, dstv, eav, rs0, rs1, rd0, rd1, pv,
               gs0, gs1, gd0, gd1):
    cid = lax.axis_index("c")
    sid = lax.axis_index("s")
    wid = cid * NS + sid
    r0 = sid * RPT
    pltpu.sync_copy(fx_hbm.at[pl.ds(r0, RPT)], fx_sh.at[pl.ds(r0, RPT)])
    pltpu.sync_copy(src_hbm.at[wid], srcv)
    pltpu.sync_copy(dst_hbm.at[wid], dstv)
    pltpu.sync_copy(ea_hbm.at[wid], eav)
    plsc.subcore_barrier()
    base = wid * EPT
    zv = jnp.zeros((16,), F32)
    perms = [lax.iota(jnp.int32, 16) ^ sh for sh in (8, 4, 2, 1)]
    rs = (rs0, rs1)
    rd = (rd0, rd1)
    gs = (gs0, gs1)
    gd = (gd0, gd1)

    def g_start(b, j):
        pltpu.async_copy(fx_sh.at[srcv.at[j]], rs[b], gs[b])
        pltpu.async_copy(fx_sh.at[dstv.at[j]], rd[b], gd[b])

    def g_wait(b, j):
        pltpu.make_async_copy(fx_sh.at[srcv.at[j]], rs[b], gs[b]).wait()
        pltpu.make_async_copy(fx_sh.at[dstv.at[j]], rd[b], gd[b]).wait()

    def compute(b, j, acc):
        @plsc.parallel_loop(0, CHK // 16, unroll=2, carry=acc)
        def edot(t, a):
            wv = eav[j, pl.ds(t * 16, 16)]
            e0 = t * 16
            for l in range(16):
                e = e0 + l
                v = (rs[b][e, pl.ds(0, 16)] * rd[b][e, pl.ds(0, 16)]
                     + rs[b][e, pl.ds(16, 16)] * rd[b][e, pl.ds(16, 16)])
                for p in perms:
                    v = v + jnp.take(v, p)
                d = v - wv[l]
                ok = (base + j * CHK + e) < E
                a = a + jnp.where(ok, d * d, zv)
            return a

        return edot

    g_start(0, 0)

    def pair(jj, acc):
        j = 2 * jj
        g_wait(0, j)
        g_start(1, j + 1)
        acc = compute(0, j, acc)
        g_wait(1, j + 1)
        g_start(0, j + 2)
        return compute(1, j + 1, acc)

    accv = lax.fori_loop(0, (NCH - 2) // 2, pair, zv)
    jL = NCH - 2
    g_wait(0, jL)
    g_start(1, jL + 1)
    accv = compute(0, jL, accv)
    g_wait(1, jL + 1)
    accv = compute(1, jL + 1, accv)
    pv[...] = accv * (1.0 / 16.0)
    pltpu.sync_copy(pv, out_hbm.at[wid])


_loss_call = pl.kernel(
    _loss_body,
    out_type=jax.ShapeDtypeStruct((NW, 16), F32),
    mesh=_mesh(),
    compiler_params=pltpu.CompilerParams(use_tc_tiling_on_sc=False),
    scratch_types=[
        pltpu.VMEM_SHARED((NP, CP), F32),
        pltpu.VMEM((NCH, CHK), jnp.int32),
        pltpu.VMEM((NCH, CHK), jnp.int32),
        pltpu.VMEM((NCH, CHK), F32),
        pltpu.VMEM((CHK, CP), F32),
        pltpu.VMEM((CHK, CP), F32),
        pltpu.VMEM((CHK, CP), F32),
        pltpu.VMEM((CHK, CP), F32),
        pltpu.VMEM((16,), F32),
        pltpu.SemaphoreType.DMA,
        pltpu.SemaphoreType.DMA,
        pltpu.SemaphoreType.DMA,
        pltpu.SemaphoreType.DMA,
    ],
)


def _tc1_body(xp_ref, w1_ref, degp_ref, hs1_ref, dis_ref):
    deg = degp_ref[0] + degp_ref[1] + 1.0
    dis = lax.rsqrt(deg)
    h1p = jnp.dot(xp_ref[...], w1_ref[...], preferred_element_type=F32)
    hs1_ref[...] = h1p * dis
    dis_ref[...] = dis


def _tc2_body(acc_ref, hs1_ref, dis_ref, b1_ref, w2_ref, g_ref, hs2_ref):
    dis = dis_ref[...]
    h1 = jnp.maximum(
        dis * (acc_ref[0] + acc_ref[1] + hs1_ref[...]) + b1_ref[...], 0.0)
    g = jnp.dot(h1, w2_ref[...], preferred_element_type=F32)
    g_ref[...] = g
    hs2_ref[...] = g * dis


def _tc3_body(acc_ref, hs2_ref, dis_ref, b2_ref, fx_ref, preg_ref):
    dis = dis_ref[...]
    h2 = dis * (acc_ref[0] + acc_ref[1] + hs2_ref[...]) + b2_ref[...]
    col = lax.broadcasted_iota(jnp.int32, (NP, CP), 1)
    row = lax.broadcasted_iota(jnp.int32, (NP, CP), 0)
    cm = col < C
    h2m = jnp.where(cm, h2, -1e30)
    m = jnp.max(h2m, axis=1, keepdims=True)
    p = jnp.exp(h2m - m)
    fx = p / jnp.sum(p, axis=1, keepdims=True)
    fx_ref[...] = fx
    nfx = jnp.where(cm & (row < N), jnp.log(1.0 - fx * fx), 0.0)
    s = jnp.sum(nfx, axis=0, keepdims=True)
    colr = lax.broadcasted_iota(jnp.int32, (1, CP), 1)
    pr = jnp.where(colr < C, jnp.log(1.0001 - jnp.exp(s)), 0.0)
    preg_ref[...] = jnp.reshape(-jnp.sum(pr), (1, 1))


def kernel(x, edge_index, edge_attr, W1, b1, W2, b2):
    src = edge_index[0]
    dst = edge_index[1]
    npad = EPAD - E
    pad_idx = (jnp.arange(npad, dtype=jnp.int32) * 37) % N
    src3 = jnp.concatenate([src, pad_idx]).reshape(NW, NCH, CHK)
    dst3 = jnp.concatenate([dst, pad_idx]).reshape(NW, NCH, CHK)
    ea3 = jnp.concatenate([edge_attr, jnp.zeros((npad,), F32)]).reshape(NW, NCH, CHK)

    xp = jnp.pad(x, ((0, NP - N), (0, 0)))
    w2p = jnp.pad(W2, ((0, 0), (0, CP - C)))
    b1r = b1.reshape(1, H)
    b2r = jnp.pad(b2, (0, CP - C)).reshape(1, CP)
    z1 = jnp.zeros((NP,), F32)
    z64 = jnp.zeros((NP, H), F32)
    z32 = jnp.zeros((NP, CP), F32)

    degp = _deg_call(dst3, ea3, z1)
    degp3 = degp.reshape(NC, NP, 1)

    hs1, dis = pl.pallas_call(
        _tc1_body,
        out_shape=[jax.ShapeDtypeStruct((NP, H), F32),
                   jax.ShapeDtypeStruct((NP, 1), F32)],
    )(xp, W1, degp3)

    acc1 = _acc64_call(hs1, src3, dst3, ea3, z64)

    g, hs2 = pl.pallas_call(
        _tc2_body,
        out_shape=[jax.ShapeDtypeStruct((NP, CP), F32),
                   jax.ShapeDtypeStruct((NP, CP), F32)],
    )(acc1, hs1, dis, b1r, w2p)

    acc2 = _acc32_call(hs2, src3, dst3, ea3, z32)

    fxp, preg = pl.pallas_call(
        _tc3_body,
        out_shape=[jax.ShapeDtypeStruct((NP, CP), F32),
                   jax.ShapeDtypeStruct((1, 1), F32)],
    )(acc2, hs2, dis, b2r)

    parts = _loss_call(fxp, src3, dst3, ea3)
    loss = jnp.sum(parts) / E + REG * preg[0, 0]
    return fxp[:N, :C], loss


# speedup vs baseline: 1.0091x; 1.0091x over previous
import jax
import jax.numpy as jnp
from jax import lax
from jax.experimental import pallas as pl
from jax.experimental.pallas import tpu as pltpu
from jax.experimental.pallas import tpu_sc as plsc

N = 10000
E = 320000
D = 128
H = 64
C = 30
CP = 32
REG = 0.01

NC = 2
NS = 16
NW = NC * NS
NP = 10240
RPT = NP // NS
EPT = 10240
EPAD = NW * EPT
NCH = 32
CHK = 320

F32 = jnp.float32


def _mesh():
    return plsc.VectorSubcoreMesh(
        core_axis_name="c", subcore_axis_name="s", num_cores=NC, num_subcores=NS
    )


def _deg_body(dst_hbm, ea_hbm, z1_hbm, out_hbm, deg_sh, idx_v, val_v):
    cid = lax.axis_index("c")
    sid = lax.axis_index("s")
    wid = cid * NS + sid
    r0 = sid * RPT
    pltpu.sync_copy(z1_hbm.at[pl.ds(r0, RPT)], deg_sh.at[pl.ds(r0, RPT)])
    pltpu.sync_copy(dst_hbm.at[wid], idx_v)
    pltpu.sync_copy(ea_hbm.at[wid], val_v)
    plsc.subcore_barrier()

    def chunk(j, carry):
        pltpu.sync_copy(val_v.at[j], deg_sh.at[idx_v.at[j]], add=True)
        return carry

    lax.fori_loop(0, NCH, chunk, 0)
    plsc.subcore_barrier()
    pltpu.sync_copy(deg_sh.at[pl.ds(r0, RPT)], out_hbm.at[cid, pl.ds(r0, RPT)])


_deg_call = pl.kernel(
    _deg_body,
    out_type=jax.ShapeDtypeStruct((NC, NP), F32),
    mesh=_mesh(),
    compiler_params=pltpu.CompilerParams(use_tc_tiling_on_sc=False),
    scratch_types=[
        pltpu.VMEM_SHARED((NP,), F32),
        pltpu.VMEM((NCH, CHK), jnp.int32),
        pltpu.VMEM((NCH, CHK), F32),
    ],
)


def _make_acc(Dw):
    nq = Dw // 16

    def body(tab_hbm, src_hbm, dst_hbm, ea_hbm, zt_hbm, out_hbm,
             acc_sh, srcv, dstv, eav, rows0, rows1, g0, g1, w0, w1):
        cid = lax.axis_index("c")
        sid = lax.axis_index("s")
        wid = cid * NS + sid
        r0 = sid * RPT
        pltpu.sync_copy(zt_hbm.at[pl.ds(r0, RPT)], acc_sh.at[pl.ds(r0, RPT)])
        pltpu.sync_copy(src_hbm.at[wid], srcv)
        pltpu.sync_copy(dst_hbm.at[wid], dstv)
        pltpu.sync_copy(ea_hbm.at[wid], eav)
        plsc.subcore_barrier()

        rows = (rows0, rows1)
        gsem = (g0, g1)
        wsem = (w0, w1)

        def g_start(b, j):
            pltpu.async_copy(tab_hbm.at[srcv.at[j]], rows[b], gsem[b])

        def g_wait(b, j):
            pltpu.make_async_copy(tab_hbm.at[srcv.at[j]], rows[b], gsem[b]).wait()

        def w_start(b, j):
            pltpu.async_copy(rows[b], acc_sh.at[dstv.at[j]], wsem[b], add=True)

        def w_wait(b, j):
            pltpu.make_async_copy(rows[b], acc_sh.at[dstv.at[j]], wsem[b]).wait()

        def scale(j, rbuf):
            @plsc.parallel_loop(0, CHK // 16, unroll=2)
            def grp(t):
                wv = eav[j, pl.ds(t * 16, 16)]
                e0 = t * 16
                for l in range(16):
                    w = wv[l]
                    for q in range(nq):
                        sl = pl.ds(q * 16, 16)
                        rbuf[e0 + l, sl] = rbuf[e0 + l, sl] * w

        pltpu.sync_copy(tab_hbm.at[srcv.at[0]], rows[0])
        scale(0, rows[0])
        w_start(0, 0)
        g_start(1, 1)

        def slot(b, j):
            g_wait(b, j)
            scale(j, rows[b])
            w_wait(1 - b, j - 1)
            w_start(b, j)
            g_start(1 - b, j + 1)

        def pair(jj, carry):
            slot(1, 2 * jj + 1)
            slot(0, 2 * jj + 2)
            return carry

        lax.fori_loop(0, (NCH - 2) // 2, pair, 0)
        jL = NCH - 1
        g_wait(1, jL)
        scale(jL, rows[1])
        w_wait(0, jL - 1)
        w_start(1, jL)
        w_wait(1, jL)
        plsc.subcore_barrier()
        pltpu.sync_copy(acc_sh.at[pl.ds(r0, RPT)], out_hbm.at[cid, pl.ds(r0, RPT)])

    return pl.kernel(
        body,
        out_type=jax.ShapeDtypeStruct((NC, NP, Dw), F32),
        mesh=_mesh(),
        compiler_params=pltpu.CompilerParams(use_tc_tiling_on_sc=False),
        scratch_types=[
            pltpu.VMEM_SHARED((NP, Dw), F32),
            pltpu.VMEM((NCH, CHK), jnp.int32),
            pltpu.VMEM((NCH, CHK), jnp.int32),
            pltpu.VMEM((NCH, CHK), F32),
            pltpu.VMEM((CHK, Dw), F32),
            pltpu.VMEM((CHK, Dw), F32),
            pltpu.SemaphoreType.DMA,
            pltpu.SemaphoreType.DMA,
            pltpu.SemaphoreType.DMA,
            pltpu.SemaphoreType.DMA,
        ],
    )


_acc64_call = _make_acc(H)
_acc32_call = _make_acc(CP)


def _loss_body(fx_hbm, src_hbm, dst_hbm, ea_hbm, out_hbm,
               fx_sh, srcv, dstv, eav, rs0, rs1, rd0, rd1, pv,
               gs0, gs1, gd0, gd1):
    cid = lax.axis_index("c")
    sid = lax.axis_index("s")
    wid = cid * NS + sid
    r0 = sid * RPT
    pltpu.sync_copy(fx_hbm.at[pl.ds(r0, RPT)], fx_sh.at[pl.ds(r0, RPT)])
    pltpu.sync_copy(src_hbm.at[wid], srcv)
    pltpu.sync_copy(dst_hbm.at[wid], dstv)
    pltpu.sync_copy(ea_hbm.at[wid], eav)
    plsc.subcore_barrier()
    base = wid * EPT
    zv = jnp.zeros((16,), F32)
    perms = [lax.iota(jnp.int32, 16) ^ sh for sh in (8, 4, 2, 1)]
    rs = (rs0, rs1)
    rd = (rd0, rd1)
    gs = (gs0, gs1)
    gd = (gd0, gd1)

    def g_start(b, j):
        pltpu.async_copy(fx_sh.at[srcv.at[j]], rs[b], gs[b])
        pltpu.async_copy(fx_sh.at[dstv.at[j]], rd[b], gd[b])

    def g_wait(b, j):
        pltpu.make_async_copy(fx_sh.at[srcv.at[j]], rs[b], gs[b]).wait()
        pltpu.make_async_copy(fx_sh.at[dstv.at[j]], rd[b], gd[b]).wait()

    def compute(b, j, acc):
        @plsc.parallel_loop(0, CHK // 16, carry=acc)
        def edot(t, a):
            wv = eav[j, pl.ds(t * 16, 16)]
            e0 = t * 16
            for l in range(16):
                e = e0 + l
                v = (rs[b][e, pl.ds(0, 16)] * rd[b][e, pl.ds(0, 16)]
                     + rs[b][e, pl.ds(16, 16)] * rd[b][e, pl.ds(16, 16)])
                for p in perms:
                    v = v + jnp.take(v, p)
                d = v - wv[l]
                ok = (base + j * CHK + e) < E
                a = a + jnp.where(ok, d * d, zv)
            return a

        return edot

    g_start(0, 0)

    def pair(jj, acc):
        j = 2 * jj
        g_wait(0, j)
        g_start(1, j + 1)
        acc = compute(0, j, acc)
        g_wait(1, j + 1)
        g_start(0, j + 2)
        return compute(1, j + 1, acc)

    accv = lax.fori_loop(0, (NCH - 2) // 2, pair, zv)
    jL = NCH - 2
    g_wait(0, jL)
    g_start(1, jL + 1)
    accv = compute(0, jL, accv)
    g_wait(1, jL + 1)
    accv = compute(1, jL + 1, accv)
    pv[...] = accv * (1.0 / 16.0)
    pltpu.sync_copy(pv, out_hbm.at[wid])


_loss_call = pl.kernel(
    _loss_body,
    out_type=jax.ShapeDtypeStruct((NW, 16), F32),
    mesh=_mesh(),
    compiler_params=pltpu.CompilerParams(use_tc_tiling_on_sc=False),
    scratch_types=[
        pltpu.VMEM_SHARED((NP, CP), F32),
        pltpu.VMEM((NCH, CHK), jnp.int32),
        pltpu.VMEM((NCH, CHK), jnp.int32),
        pltpu.VMEM((NCH, CHK), F32),
        pltpu.VMEM((CHK, CP), F32),
        pltpu.VMEM((CHK, CP), F32),
        pltpu.VMEM((CHK, CP), F32),
        pltpu.VMEM((CHK, CP), F32),
        pltpu.VMEM((16,), F32),
        pltpu.SemaphoreType.DMA,
        pltpu.SemaphoreType.DMA,
        pltpu.SemaphoreType.DMA,
        pltpu.SemaphoreType.DMA,
    ],
)


def _tc1_body(xp_ref, w1_ref, degp_ref, hs1_ref, dis_ref):
    deg = degp_ref[0] + degp_ref[1] + 1.0
    dis = lax.rsqrt(deg)
    h1p = jnp.dot(xp_ref[...], w1_ref[...], preferred_element_type=F32)
    hs1_ref[...] = h1p * dis
    dis_ref[...] = dis


def _tc2_body(acc_ref, hs1_ref, dis_ref, b1_ref, w2_ref, g_ref, hs2_ref):
    dis = dis_ref[...]
    h1 = jnp.maximum(
        dis * (acc_ref[0] + acc_ref[1] + hs1_ref[...]) + b1_ref[...], 0.0)
    g = jnp.dot(h1, w2_ref[...], preferred_element_type=F32)
    g_ref[...] = g
    hs2_ref[...] = g * dis


def _tc3_body(acc_ref, hs2_ref, dis_ref, b2_ref, fx_ref, preg_ref):
    dis = dis_ref[...]
    h2 = dis * (acc_ref[0] + acc_ref[1] + hs2_ref[...]) + b2_ref[...]
    col = lax.broadcasted_iota(jnp.int32, (NP, CP), 1)
    row = lax.broadcasted_iota(jnp.int32, (NP, CP), 0)
    cm = col < C
    h2m = jnp.where(cm, h2, -1e30)
    m = jnp.max(h2m, axis=1, keepdims=True)
    p = jnp.exp(h2m - m)
    fx = p / jnp.sum(p, axis=1, keepdims=True)
    fx_ref[...] = fx
    nfx = jnp.where(cm & (row < N), jnp.log(1.0 - fx * fx), 0.0)
    s = jnp.sum(nfx, axis=0, keepdims=True)
    colr = lax.broadcasted_iota(jnp.int32, (1, CP), 1)
    pr = jnp.where(colr < C, jnp.log(1.0001 - jnp.exp(s)), 0.0)
    preg_ref[...] = jnp.reshape(-jnp.sum(pr), (1, 1))


def kernel(x, edge_index, edge_attr, W1, b1, W2, b2):
    src = edge_index[0]
    dst = edge_index[1]
    npad = EPAD - E
    pad_idx = (jnp.arange(npad, dtype=jnp.int32) * 37) % N
    src3 = jnp.concatenate([src, pad_idx]).reshape(NW, NCH, CHK)
    dst3 = jnp.concatenate([dst, pad_idx]).reshape(NW, NCH, CHK)
    ea3 = jnp.concatenate([edge_attr, jnp.zeros((npad,), F32)]).reshape(NW, NCH, CHK)

    xp = jnp.pad(x, ((0, NP - N), (0, 0)))
    w2p = jnp.pad(W2, ((0, 0), (0, CP - C)))
    b1r = b1.reshape(1, H)
    b2r = jnp.pad(b2, (0, CP - C)).reshape(1, CP)
    z1 = jnp.zeros((NP,), F32)
    z64 = jnp.zeros((NP, H), F32)
    z32 = jnp.zeros((NP, CP), F32)

    degp = _deg_call(dst3, ea3, z1)
    degp3 = degp.reshape(NC, NP, 1)

    hs1, dis = pl.pallas_call(
        _tc1_body,
        out_shape=[jax.ShapeDtypeStruct((NP, H), F32),
                   jax.ShapeDtypeStruct((NP, 1), F32)],
    )(xp, W1, degp3)

    acc1 = _acc64_call(hs1, src3, dst3, ea3, z64)

    g, hs2 = pl.pallas_call(
        _tc2_body,
        out_shape=[jax.ShapeDtypeStruct((NP, CP), F32),
                   jax.ShapeDtypeStruct((NP, CP), F32)],
    )(acc1, hs1, dis, b1r, w2p)

    acc2 = _acc32_call(hs2, src3, dst3, ea3, z32)

    fxp, preg = pl.pallas_call(
        _tc3_body,
        out_shape=[jax.ShapeDtypeStruct((NP, CP), F32),
                   jax.ShapeDtypeStruct((1, 1), F32)],
    )(acc2, hs2, dis, b2r)

    parts = _loss_call(fxp, src3, dst3, ea3)
    loss = jnp.sum(parts) / E + REG * preg[0, 0]
    return fxp[:N, :C], loss
